# baseline (speedup 1.0000x reference)
import functools

import jax
import jax.numpy as jnp
from jax import lax
from jax.experimental import pallas as pl
from jax.experimental.pallas import tpu as pltpu
from jax.experimental.pallas import tpu_sc as plsc

NUM_CORES = 2
NUM_SUBCORES = 16
NW = NUM_CORES * NUM_SUBCORES

BATCH = 4096
SEQ = 50
D = 64
B = BATCH * SEQ
SPLIT = 50000
BPW = B // NW
CHUNK = 104
MAXCH = (BPW + CHUNK - 1) // CHUNK + 1
NGROUP = 400

RING = 6
AHEAD = 3
NITER = ((MAXCH + RING - 1) // RING) * RING

_mesh = plsc.VectorSubcoreMesh(core_axis_name="c", subcore_axis_name="s")


@functools.partial(
    pl.kernel,
    out_type=jax.ShapeDtypeStruct((B + 8, D), jnp.float32),
    mesh=_mesh,
    scratch_types=[
        pltpu.VMEM((BPW,), jnp.int32),
        pltpu.VMEM((MAXCH, CHUNK), jnp.int32),
        pltpu.VMEM((MAXCH, CHUNK), jnp.int32),
        pltpu.VMEM((MAXCH, CHUNK), jnp.int32),
        pltpu.VMEM((MAXCH, CHUNK), jnp.int32),
        pltpu.VMEM((RING, CHUNK, D), jnp.float32),
        [pltpu.SemaphoreType.DMA] * RING,
        [pltpu.SemaphoreType.DMA] * RING,
    ],
    compiler_params=pltpu.CompilerParams(
        use_tc_tiling_on_sc=False, needs_layout_passes=False
    ),
)
def _gather_kernel(t0, t1, xf, out, idx_v, idx0b, pos0b, idx1b, pos1b,
                   rows_v, gsems, wsems):
    wid = lax.axis_index("s") * NUM_CORES + lax.axis_index("c")
    wbase = wid * BPW
    iota = lax.iota(jnp.int32, 16)

    pltpu.sync_copy(xf.at[pl.ds(wbase, BPW)], idx_v)

    def compress(i, carry):
        c0, c1 = carry
        v = idx_v[pl.ds(pl.multiple_of(i * 16, 16), 16)]
        posv = wbase + i * 16 + iota
        m0 = v < SPLIT
        e = plsc.cumsum(m0.astype(jnp.int32))
        cnt0 = jnp.sum(m0.astype(jnp.int32))
        p0 = c0 + e - 1
        r0 = p0 // CHUNK
        q0 = p0 - r0 * CHUNK
        plsc.store_scatter(idx0b, [r0, q0], v, mask=m0)
        plsc.store_scatter(pos0b, [r0, q0], posv, mask=m0)
        m1 = jnp.logical_not(m0)
        e1 = (iota + 1) - e
        p1 = c1 + e1 - 1
        r1 = p1 // CHUNK
        q1 = p1 - r1 * CHUNK
        plsc.store_scatter(idx1b, [r1, q1], v - SPLIT, mask=m1)
        plsc.store_scatter(pos1b, [r1, q1], posv, mask=m1)
        return c0 + cnt0, c1 + (16 - cnt0)

    zero = jnp.int32(0)
    c0, c1 = lax.fori_loop(0, NGROUP, compress, (zero, zero))

    zeros16 = jnp.zeros((16,), jnp.int32)
    trash16 = jnp.full((16,), B, jnp.int32)
    for k in range(7):
        for c, ib, pb in ((c0, idx0b, pos0b), (c1, idx1b, pos1b)):
            p = c + iota + k * 16
            r = p // CHUNK
            q = p - r * CHUNK
            plsc.store_scatter(ib, [r, q], zeros16)
            plsc.store_scatter(pb, [r, q], trash16)

    n0 = (c0 + CHUNK - 1) // CHUNK
    ntot = n0 + (c1 + CHUNK - 1) // CHUNK

    plsc.subcore_barrier()

    def start_gather(j, slot):
        @pl.when(j < n0)
        def _():
            pltpu.async_copy(t0.at[idx0b.at[j]], rows_v.at[slot], gsems[slot])

        @pl.when(jnp.logical_and(j >= n0, j < ntot))
        def _():
            pltpu.async_copy(t1.at[idx1b.at[j - n0]], rows_v.at[slot],
                             gsems[slot])

    def wait_gather(j, slot):
        @pl.when(j < ntot)
        def _():
            pltpu.make_async_copy(t0.at[idx0b.at[0]], rows_v.at[slot],
                                  gsems[slot]).wait()

    def start_scatter(j, slot):
        @pl.when(j < n0)
        def _():
            pltpu.async_copy(rows_v.at[slot], out.at[pos0b.at[j]],
                             wsems[slot])

        @pl.when(jnp.logical_and(j >= n0, j < ntot))
        def _():
            pltpu.async_copy(rows_v.at[slot], out.at[pos1b.at[j - n0]],
                             wsems[slot])

    def wait_scatter(j, slot):
        @pl.when(jnp.logical_and(j >= 0, j < ntot))
        def _():
            pltpu.make_async_copy(rows_v.at[slot], out.at[pos0b.at[0]],
                                  wsems[slot]).wait()

    for k in range(AHEAD):
        start_gather(jnp.int32(k), k)

    def outer(g, carry):
        for b in range(RING):
            j = g * RING + b
            wait_gather(j, b)
            start_scatter(j, b)
            wait_scatter(j + AHEAD - RING, (b + AHEAD) % RING)
            start_gather(j + AHEAD, (b + AHEAD) % RING)
        return carry

    lax.fori_loop(0, NITER // RING, outer, 0)


def kernel(x, W_frame_0, W_frame_1):
    out = _gather_kernel(W_frame_0, W_frame_1, x.reshape(-1))
    return out[:B].reshape(BATCH, SEQ, D)
